# Initial kernel scaffold; baseline (speedup 1.0000x reference)
import jax
import jax.numpy as jnp
from jax.experimental import pallas as pl


def kernel(x, edge_index, W0, b0, W1, b1, W2, b2):
    raise NotImplementedError("write your pallas kernel here")


# speedup vs baseline: 1.9331x; 1.9331x over previous
import functools

import jax
import jax.numpy as jnp
from jax import lax
from jax.experimental import pallas as pl
from jax.experimental.pallas import tpu as pltpu
from jax.experimental.pallas import tpu_sc as plsc

N = 10000
E = 320000
GR = 128
NW = 32
DPW = 320
NPAD = NW * DPW
TRASH = DPW
ECHUNK = 2560
NJ = ECHUNK // 16
NPCHUNK = E // ECHUNK
SEG = 128
STAGE = ECHUNK + 16
CAP = 324608


def _mesh():
    return plsc.VectorSubcoreMesh(core_axis_name="c", subcore_axis_name="s")


def _worker_id():
    return lax.axis_index("s") * 2 + lax.axis_index("c")


def _partition_body(src_hbm, dst_hbm, ls_hbm, ld_hbm, cnt_hbm,
                    sbuf, dbuf, stage_s, stage_d, cvec):
    w = _worker_id()
    lo = w * DPW
    zero16 = jnp.zeros((16,), jnp.int32)
    trash16 = jnp.full((16,), TRASH, jnp.int32)

    def chunk_body(g, cnt):
        pltpu.sync_copy(dst_hbm.at[pl.ds(g * ECHUNK, ECHUNK)], dbuf)
        pltpu.sync_copy(src_hbm.at[pl.ds(g * ECHUNK, ECHUNK)], sbuf)

        def j_body(j, cl):
            vd = dbuf[pl.ds(j * 16, 16)]
            vs = sbuf[pl.ds(j * 16, 16)]
            m = (vd >= lo) & (vd < lo + DPW)
            pos = plsc.cumsum(jnp.where(m, 1, 0))
            idx = (cl - 1) + pos
            plsc.store_scatter(stage_s, [idx], vs, mask=m)
            plsc.store_scatter(stage_d, [idx], vd - lo, mask=m)
            return cl + pos[15]

        cl = lax.fori_loop(0, NJ, j_body, 0)
        stage_s[pl.ds(cl, 16)] = zero16
        stage_d[pl.ds(cl, 16)] = trash16
        off = pl.multiple_of(w * CAP + cnt, 16)
        pltpu.sync_copy(stage_s, ls_hbm.at[pl.ds(off, STAGE)])
        pltpu.sync_copy(stage_d, ld_hbm.at[pl.ds(off, STAGE)])
        return cnt + ((cl + 15) // 16) * 16

    cnt = lax.fori_loop(0, NPCHUNK, chunk_body, 0)

    def pad_body(j, _):
        stage_s[pl.ds(j * 16, 16)] = zero16
        stage_d[pl.ds(j * 16, 16)] = trash16
        return 0

    lax.fori_loop(0, SEG // 16, pad_body, 0)
    off = pl.multiple_of(w * CAP + cnt, 16)
    pltpu.sync_copy(stage_s.at[pl.ds(0, SEG)], ls_hbm.at[pl.ds(off, SEG)])
    pltpu.sync_copy(stage_d.at[pl.ds(0, SEG)], ld_hbm.at[pl.ds(off, SEG)])
    cvec[...] = jnp.full((16,), 0, jnp.int32) + (cnt + SEG - 1) // SEG
    pltpu.sync_copy(cvec, cnt_hbm.at[pl.ds(pl.multiple_of(w * 16, 16), 16)])


def _partition(src, dst):
    f = pl.kernel(
        _partition_body,
        out_type=[
            jax.ShapeDtypeStruct((NW * CAP,), jnp.int32),
            jax.ShapeDtypeStruct((NW * CAP,), jnp.int32),
            jax.ShapeDtypeStruct((NW * 16,), jnp.int32),
        ],
        mesh=_mesh(),
        compiler_params=pltpu.CompilerParams(needs_layout_passes=False),
        scratch_types=[
            pltpu.VMEM((ECHUNK,), jnp.int32),
            pltpu.VMEM((ECHUNK,), jnp.int32),
            pltpu.VMEM((STAGE,), jnp.int32),
            pltpu.VMEM((STAGE,), jnp.int32),
            pltpu.VMEM((16,), jnp.int32),
        ],
    )
    return f(src, dst)


def _segmax_body(a_hbm, b_hbm, ls_hbm, ld_hbm, cnt_hbm, out_hbm,
                 acc, arow, rows, sidx, dloc, cvec, sem):
    w = _worker_id()
    lo = w * DPW
    neg = jnp.full((16,), float("-inf"), jnp.float32)

    def init_r(r, _):
        for v in range(8):
            acc[r, pl.ds(v * 16, 16)] = neg
        return 0

    lax.fori_loop(0, DPW + 1, init_r, 0)

    pltpu.sync_copy(cnt_hbm.at[pl.ds(pl.multiple_of(w * 16, 16), 16)], cvec)
    nch = cvec[...][0]

    def g_body(g, _):
        pltpu.sync_copy(ls_hbm.at[pl.ds(pl.multiple_of(w * CAP + g * SEG, 16), SEG)], sidx)
        pltpu.sync_copy(ld_hbm.at[pl.ds(pl.multiple_of(w * CAP + g * SEG, 16), SEG)], dloc.at[pl.ds(0, SEG)])
        pltpu.async_copy(b_hbm.at[sidx], rows, sem).wait()

        def k_body(k, _):
            dl = dloc[pl.ds(k, 16)][0]
            for v in range(8):
                sl = pl.ds(v * 16, 16)
                acc[dl, sl] = jnp.maximum(acc[dl, sl], rows[k, sl])
            return 0

        lax.fori_loop(0, SEG, k_body, 0)
        return 0

    lax.fori_loop(0, nch, g_body, 0)

    pltpu.sync_copy(a_hbm.at[pl.ds(pl.multiple_of(lo, 8), DPW)], arow)

    def fin_r(r, _):
        for v in range(8):
            sl = pl.ds(v * 16, 16)
            acc[r, sl] = jnp.maximum(acc[r, sl] + arow[r, sl], 0.0)
        return 0

    lax.fori_loop(0, DPW, fin_r, 0)
    pltpu.sync_copy(acc.at[pl.ds(0, DPW)], out_hbm.at[pl.ds(pl.multiple_of(lo, 8), DPW)])


def _segmax(A, B, ls, ld, cnts):
    f = pl.kernel(
        _segmax_body,
        out_type=jax.ShapeDtypeStruct((NPAD, GR), jnp.float32),
        mesh=_mesh(),
        compiler_params=pltpu.CompilerParams(needs_layout_passes=False),
        scratch_types=[
            pltpu.VMEM((DPW + 1, GR), jnp.float32),
            pltpu.VMEM((DPW, GR), jnp.float32),
            pltpu.VMEM((SEG, GR), jnp.float32),
            pltpu.VMEM((SEG,), jnp.int32),
            pltpu.VMEM((SEG + 16,), jnp.int32),
            pltpu.VMEM((16,), jnp.int32),
            pltpu.SemaphoreType.DMA,
        ],
    )
    return f(A, B, ls, ld, cnts)


def _mm_block(h_ref, wd_ref, wb_ref, bias_ref, a_ref, b_ref):
    hb = h_ref[...]
    a_ref[...] = (jnp.dot(hb, wd_ref[...], preferred_element_type=jnp.float32)
                  + bias_ref[...])
    b_ref[...] = jnp.dot(hb, wb_ref[...], preferred_element_type=jnp.float32)


def _tc_mm(h, Wd, Wb, bias):
    M, cin = h.shape
    BM = 1024
    return pl.pallas_call(
        _mm_block,
        grid=(M // BM,),
        in_specs=[
            pl.BlockSpec((BM, cin), lambda i: (i, 0)),
            pl.BlockSpec((cin, GR), lambda i: (0, 0)),
            pl.BlockSpec((cin, GR), lambda i: (0, 0)),
            pl.BlockSpec((1, GR), lambda i: (0, 0)),
        ],
        out_specs=[
            pl.BlockSpec((BM, GR), lambda i: (i, 0)),
            pl.BlockSpec((BM, GR), lambda i: (i, 0)),
        ],
        out_shape=[
            jax.ShapeDtypeStruct((M, GR), jnp.float32),
            jax.ShapeDtypeStruct((M, GR), jnp.float32),
        ],
    )(h, Wd, Wb, bias)


def _final_block(x_ref, c0_ref, c1_ref, c2_ref, o_ref):
    for t, r in enumerate((x_ref, c0_ref, c1_ref, c2_ref)):
        v = r[...]
        p = jnp.maximum(jnp.maximum(v[:, 0:32], v[:, 32:64]),
                        jnp.maximum(v[:, 64:96], v[:, 96:128]))
        o_ref[:, t * 32:(t + 1) * 32] = p


def _tc_final(x, c0, c1, c2):
    BM = 2000
    return pl.pallas_call(
        _final_block,
        grid=(N // BM,),
        in_specs=[pl.BlockSpec((BM, GR), lambda i: (i, 0))] * 4,
        out_specs=pl.BlockSpec((BM, GR), lambda i: (i, 0)),
        out_shape=jax.ShapeDtypeStruct((N, GR), jnp.float32),
    )(x, c0, c1, c2)


_PERM = tuple(4 * (j % 32) + j // 32 for j in range(128))


def _permute_rows(Wpart):
    import numpy as np
    secs = [Wpart[0:GR]]
    perm = np.array(_PERM)
    for s in range(1, Wpart.shape[0] // GR):
        secs.append(Wpart[GR * s:GR * (s + 1)][perm])
    return jnp.concatenate(secs, axis=0)


def kernel(x, edge_index, W0, b0, W1, b1, W2, b2):
    import numpy as np
    perm = np.array(_PERM)
    src = edge_index[0].astype(jnp.int32)
    dst = edge_index[1].astype(jnp.int32)
    xp = jnp.pad(x, ((0, NPAD - N), (0, 0)))
    ls, ld, cnts = _partition(src, dst)
    h = xp
    cs = []
    for W, b in ((W0, b0), (W1, b1), (W2, b2)):
        cin = h.shape[1]
        Wd = _permute_rows((W[:cin] - W[cin:]))[:, perm]
        Wb = _permute_rows(W[cin:])[:, perm]
        A, B = _tc_mm(h, Wd, Wb, b[perm].reshape(1, GR))
        c = _segmax(A, B, ls, ld, cnts)
        cs.append(c)
        h = jnp.concatenate([h, c], axis=-1)
    return _tc_final(x[:, perm], cs[0], cs[1], cs[2])
